# baseline (speedup 1.0000x reference)
import jax
import jax.numpy as jnp
from jax.experimental import pallas as pl
from jax.experimental.pallas import tpu as pltpu

NUM_CHUNKS = 8


def _dma_copy(preds_ref, out_ref, sems):
    b = preds_ref.shape[0]
    rows = b // NUM_CHUNKS
    for i in range(NUM_CHUNKS):
        pltpu.make_async_copy(
            preds_ref.at[pl.ds(i * rows, rows), :],
            out_ref.at[pl.ds(i * rows, rows), :],
            sems.at[i],
        ).start()
    for i in range(NUM_CHUNKS):
        pltpu.make_async_copy(
            preds_ref.at[pl.ds(i * rows, rows), :],
            out_ref.at[pl.ds(i * rows, rows), :],
            sems.at[i],
        ).wait()


def kernel(preds, atoms):
    del atoms
    return pl.pallas_call(
        _dma_copy,
        in_specs=[pl.BlockSpec(memory_space=pltpu.MemorySpace.HBM)],
        out_specs=pl.BlockSpec(memory_space=pltpu.MemorySpace.HBM),
        out_shape=jax.ShapeDtypeStruct(preds.shape, preds.dtype),
        scratch_shapes=[pltpu.SemaphoreType.DMA((NUM_CHUNKS,))],
    )(preds)


# speedup vs baseline: 6.2250x; 6.2250x over previous
import jax
import jax.numpy as jnp
from jax import lax
from jax.experimental import pallas as pl
from jax.experimental.pallas import tpu as pltpu
from jax.experimental.pallas import tpu_sc as plsc

BATCH = 1024
CLASSES = 100000
TOTAL = BATCH * CLASSES
NUM_CORES = 2
NUM_SUBCORES = 16
NUM_WORKERS = NUM_CORES * NUM_SUBCORES
PER_WORKER = TOTAL // NUM_WORKERS
CHUNK = 25000
NUM_CHUNKS = PER_WORKER // CHUNK


def _sc_copy(preds_hbm, out_hbm, buf):
    wid = lax.axis_index("s") * NUM_CORES + lax.axis_index("c")
    base = wid * PER_WORKER

    def body(i, carry):
        off = base + i * CHUNK
        pltpu.sync_copy(preds_hbm.at[pl.ds(off, CHUNK)], buf)
        pltpu.sync_copy(buf, out_hbm.at[pl.ds(off, CHUNK)])
        return carry

    lax.fori_loop(0, NUM_CHUNKS, body, 0)


def kernel(preds, atoms):
    del atoms
    flat = preds.reshape(TOTAL)
    out = pl.kernel(
        _sc_copy,
        out_type=jax.ShapeDtypeStruct((TOTAL,), preds.dtype),
        mesh=plsc.VectorSubcoreMesh(core_axis_name="c", subcore_axis_name="s"),
        scratch_types=[pltpu.VMEM((CHUNK,), jnp.float32)],
    )(flat)
    return out.reshape(preds.shape)


# speedup vs baseline: 6.4685x; 1.0391x over previous
import jax
import jax.numpy as jnp
from jax import lax
from jax.experimental import pallas as pl
from jax.experimental.pallas import tpu as pltpu
from jax.experimental.pallas import tpu_sc as plsc

BATCH = 1024
CLASSES = 100000
TOTAL = BATCH * CLASSES
NUM_CORES = 2
NUM_SUBCORES = 16
NUM_WORKERS = NUM_CORES * NUM_SUBCORES
PER_WORKER = TOTAL // NUM_WORKERS
NBUF = 2
CHUNK = 50000
NUM_CHUNKS = PER_WORKER // CHUNK
NUM_GROUPS = NUM_CHUNKS // NBUF


def _sc_copy(preds_hbm, out_hbm, buf0, buf1, si0, si1, so0, so1):
    wid = lax.axis_index("s") * NUM_CORES + lax.axis_index("c")
    base = wid * PER_WORKER
    bufs = (buf0, buf1)
    in_sems = (si0, si1)
    out_sems = (so0, so1)

    def body(g, carry):
        for b in range(NBUF):
            off = base + (g * NBUF + b) * CHUNK

            @pl.when(g > 0)
            def _():
                pltpu.make_async_copy(
                    bufs[b], out_hbm.at[pl.ds(off, CHUNK)], out_sems[b]
                ).wait()

            pltpu.async_copy(
                preds_hbm.at[pl.ds(off, CHUNK)], bufs[b], in_sems[b]
            )
        for b in range(NBUF):
            off = base + (g * NBUF + b) * CHUNK
            pltpu.make_async_copy(
                preds_hbm.at[pl.ds(off, CHUNK)], bufs[b], in_sems[b]
            ).wait()
            pltpu.async_copy(
                bufs[b], out_hbm.at[pl.ds(off, CHUNK)], out_sems[b]
            )
        return carry

    lax.fori_loop(0, NUM_GROUPS, body, 0)
    for b in range(NBUF):
        pltpu.make_async_copy(
            bufs[b], out_hbm.at[pl.ds(base, CHUNK)], out_sems[b]
        ).wait()


def kernel(preds, atoms):
    del atoms
    flat = preds.reshape(TOTAL)
    out = pl.kernel(
        _sc_copy,
        out_type=jax.ShapeDtypeStruct((TOTAL,), preds.dtype),
        mesh=plsc.VectorSubcoreMesh(core_axis_name="c", subcore_axis_name="s"),
        scratch_types=[
            pltpu.VMEM((CHUNK,), jnp.float32),
            pltpu.VMEM((CHUNK,), jnp.float32),
            pltpu.SemaphoreType.DMA,
            pltpu.SemaphoreType.DMA,
            pltpu.SemaphoreType.DMA,
            pltpu.SemaphoreType.DMA,
        ],
    )(flat)
    return out.reshape(preds.shape)
